# Initial kernel scaffold; baseline (speedup 1.0000x reference)
import functools

import jax
import jax.numpy as jnp
from jax import lax
from jax.experimental import pallas as pl
from jax.experimental.pallas import tpu as pltpu
from jax.experimental.pallas import tpu_sc as plsc

D = 256
K = 8192
R = 512
CH = 1024
NCH = K // CH
GW = 128


def _argmin_body(x_ref, emb_ref, ids_ref, dsum_ref, et_ref, neg_ref):
    i = pl.program_id(0)
    xb = x_ref[...]
    x2 = jnp.sum(xb * xb, axis=1, keepdims=True)

    m = None
    for c in range(NCH):
        e = emb_ref[:, pl.ds(c * CH, CH)]
        mm = jnp.dot(xb, e, preferred_element_type=jnp.float32)
        e2 = jnp.sum(e * e, axis=0, keepdims=True)
        neg = -((x2 - 2.0 * mm) + e2)
        neg_ref[:, pl.ds(c * CH, CH)] = neg
        m = neg if m is None else jnp.maximum(m, neg)
    best = jnp.max(m, axis=1, keepdims=True)

    big = jnp.int32(2 ** 30)
    idxm = jnp.full((R, CH), big, jnp.int32)
    for c in range(NCH):
        neg = neg_ref[:, pl.ds(c * CH, CH)]
        gids = lax.broadcasted_iota(jnp.int32, (R, CH), 1) + jnp.int32(c * CH)
        idxm = jnp.minimum(idxm, jnp.where(neg == best, gids, big))
    ids = jnp.min(idxm, axis=1)

    ids_ref[0, 0, :] = ids
    dsum_ref[0, 0] = jnp.sum(x2 - best)

    @pl.when(i == 0)
    def _():
        et_ref[...] = emb_ref[...].T


def _argmin_call(xf, embed):
    nr = xf.shape[0] // R
    return pl.pallas_call(
        _argmin_body,
        grid=(nr,),
        in_specs=[
            pl.BlockSpec((R, D), lambda i: (i, 0)),
            pl.BlockSpec((D, K), lambda i: (0, 0)),
        ],
        out_specs=[
            pl.BlockSpec((1, 1, R), lambda i: (i, 0, 0)),
            pl.BlockSpec((1, 1), lambda i: (i, 0), memory_space=pltpu.SMEM),
            pl.BlockSpec((K, D), lambda i: (0, 0)),
        ],
        out_shape=[
            jax.ShapeDtypeStruct((nr, 1, R), jnp.int32),
            jax.ShapeDtypeStruct((nr, 1), jnp.float32),
            jax.ShapeDtypeStruct((K, D), jnp.float32),
        ],
        scratch_shapes=[pltpu.VMEM((R, K), jnp.float32)],
    )(xf, embed)


def _gather_call(et, ids):
    ntok = ids.shape[0]
    idx2 = ids.reshape(1, ntok)
    mesh = plsc.VectorSubcoreMesh(core_axis_name="core",
                                  subcore_axis_name="subcore")

    @functools.partial(
        pl.kernel,
        out_type=jax.ShapeDtypeStruct((ntok, D), jnp.float32),
        mesh=mesh,
    )
    def k(table_hbm, i_hbm, o_hbm):
        def body(i_vmem, o_vmem):
            pltpu.sync_copy(table_hbm.at[i_vmem.at[0]], o_vmem)

        pltpu.emit_pipeline(
            body,
            grid=(ntok // GW,),
            in_specs=[pl.BlockSpec((1, GW), index_map=lambda i: (0, i))],
            out_specs=[pl.BlockSpec((GW, D), index_map=lambda i: (i, 0))],
            core_axis_name=("core", "subcore"),
            dimension_semantics=(pltpu.PARALLEL,),
        )(i_hbm, o_hbm)

    return k(et, idx2)


def kernel(x, embed):
    ntok = x.shape[0] * x.shape[1] * x.shape[2]
    xf = x.reshape(ntok, D)
    ids3, dparts, et = _argmin_call(xf, embed)
    ids = ids3.reshape(ntok)
    q = _gather_call(et, ids).reshape(x.shape)
    diff = jnp.sum(dparts) / jnp.float32(ntok * D)
    emd_id = ids3.reshape(x.shape[:-1])
    return (q, diff, emd_id)


# speedup vs baseline: 1.1771x; 1.1771x over previous
import functools

import jax
import jax.numpy as jnp
from jax import lax
from jax.experimental import pallas as pl
from jax.experimental.pallas import tpu as pltpu
from jax.experimental.pallas import tpu_sc as plsc

D = 256
K = 8192
R = 512
CH = 1024
NCH = K // CH
GW = 128


def _argmin_body(x_ref, emb_ref, ids_ref, dsum_ref, et_ref, neg_ref):
    i = pl.program_id(0)
    xb = x_ref[...]
    x2 = jnp.sum(xb * xb, axis=1, keepdims=True)

    m = None
    for c in range(NCH):
        e = emb_ref[:, pl.ds(c * CH, CH)]
        mm = jnp.dot(xb.astype(jnp.bfloat16), e.astype(jnp.bfloat16),
                     preferred_element_type=jnp.float32)
        e2 = jnp.sum(e * e, axis=0, keepdims=True)
        neg = -((x2 - 2.0 * mm) + e2)
        neg_ref[:, pl.ds(c * CH, CH)] = neg
        m = neg if m is None else jnp.maximum(m, neg)
    best = jnp.max(m, axis=1, keepdims=True)

    big = jnp.int32(2 ** 30)
    idxm = jnp.full((R, CH), big, jnp.int32)
    for c in range(NCH):
        neg = neg_ref[:, pl.ds(c * CH, CH)]
        gids = lax.broadcasted_iota(jnp.int32, (R, CH), 1) + jnp.int32(c * CH)
        idxm = jnp.minimum(idxm, jnp.where(neg == best, gids, big))
    ids = jnp.min(idxm, axis=1)

    ids_ref[0, 0, :] = ids
    prev = jnp.where(i == 0, jnp.float32(0.0), dsum_ref[0, 0])
    dsum_ref[0, 0] = prev - jnp.sum(best)

    @pl.when(i == 0)
    def _():
        et_ref[...] = emb_ref[...].T


def _argmin_call(xf, embed):
    nr = xf.shape[0] // R
    return pl.pallas_call(
        _argmin_body,
        grid=(nr,),
        in_specs=[
            pl.BlockSpec((R, D), lambda i: (i, 0)),
            pl.BlockSpec((D, K), lambda i: (0, 0)),
        ],
        out_specs=[
            pl.BlockSpec((1, 1, R), lambda i: (i, 0, 0)),
            pl.BlockSpec((1, 1), lambda i: (0, 0), memory_space=pltpu.SMEM),
            pl.BlockSpec((K, D), lambda i: (0, 0)),
        ],
        out_shape=[
            jax.ShapeDtypeStruct((nr, 1, R), jnp.int32),
            jax.ShapeDtypeStruct((1, 1), jnp.float32),
            jax.ShapeDtypeStruct((K, D), jnp.float32),
        ],
        scratch_shapes=[pltpu.VMEM((R, K), jnp.float32)],
    )(xf, embed)


def _gather_call(et, ids):
    ntok = ids.shape[0]
    idx2 = ids.reshape(1, ntok)
    mesh = plsc.VectorSubcoreMesh(core_axis_name="core",
                                  subcore_axis_name="subcore")

    @functools.partial(
        pl.kernel,
        out_type=jax.ShapeDtypeStruct((ntok, D), jnp.float32),
        mesh=mesh,
    )
    def k(table_hbm, i_hbm, o_hbm):
        def body(i_vmem, o_vmem):
            pltpu.sync_copy(table_hbm.at[i_vmem.at[0]], o_vmem)

        pltpu.emit_pipeline(
            body,
            grid=(ntok // GW,),
            in_specs=[pl.BlockSpec((1, GW), index_map=lambda i: (0, i))],
            out_specs=[pl.BlockSpec((GW, D), index_map=lambda i: (i, 0))],
            core_axis_name=("core", "subcore"),
            dimension_semantics=(pltpu.PARALLEL,),
        )(i_hbm, o_hbm)

    return k(et, idx2)


def kernel(x, embed):
    ntok = x.shape[0] * x.shape[1] * x.shape[2]
    xf = x.reshape(ntok, D)
    ids3, dparts, et = _argmin_call(xf, embed)
    ids = ids3.reshape(ntok)
    q = _gather_call(et, ids).reshape(x.shape)
    diff = dparts[0, 0] / jnp.float32(ntok * D)
    emd_id = ids3.reshape(x.shape[:-1])
    return (q, diff, emd_id)
